# initial kernel scaffold (unmeasured)
import jax
import jax.numpy as jnp
from jax import lax
from jax.experimental import pallas as pl
from jax.experimental.pallas import tpu as pltpu

N_DEV = 4


def _mlp_body(x_ref, w1_ref, w2_ref, out_ref):
    m = x_ref.shape[0]
    blk = m // N_DEV
    for c in range(N_DEV):
        xc = x_ref[pl.ds(c * blk, blk), :]
        h = jnp.dot(xc, w1_ref[:, :], preferred_element_type=jnp.float32)
        h = jnp.maximum(h, 0.0).astype(jnp.bfloat16)
        p = jnp.dot(h, w2_ref[:, :], preferred_element_type=jnp.float32)
        out_ref[pl.ds(c * blk, blk), :] = p.astype(jnp.bfloat16)


def _mlp(x, w1, w2):
    m = x.shape[0]
    n = w2.shape[1]
    return pl.pallas_call(
        _mlp_body,
        out_shape=jax.ShapeDtypeStruct((m, n), jnp.bfloat16),
        in_specs=[pl.BlockSpec(memory_space=pltpu.VMEM)] * 3,
        out_specs=pl.BlockSpec(memory_space=pltpu.VMEM),
    )(x, w1, w2)


def _allreduce_body(
    p_ref,
    out_ref,
    rs_recv,
    stage,
    ag_recv,
    rs_send_sem,
    rs_recv_sem,
    ag_send_sem,
    ag_recv_sem,
):
    my = lax.axis_index("i")
    left = lax.rem(my + N_DEV - 1, N_DEV)
    right = lax.rem(my + 1, N_DEV)
    blk = p_ref.shape[0] // N_DEV

    barrier = pltpu.get_barrier_semaphore()
    for nbr in (left, right):
        pl.semaphore_signal(
            barrier, inc=1, device_id=(nbr,), device_id_type=pl.DeviceIdType.MESH
        )
    pl.semaphore_wait(barrier, 2)

    stage[0, :, :] = p_ref[pl.ds(my * blk, blk), :]
    for s in range(N_DEV - 1):
        rdma = pltpu.make_async_remote_copy(
            src_ref=stage.at[s],
            dst_ref=rs_recv.at[s],
            send_sem=rs_send_sem.at[s],
            recv_sem=rs_recv_sem.at[s],
            device_id=(right,),
            device_id_type=pl.DeviceIdType.MESH,
        )
        rdma.start()
        rdma.wait()
        c = lax.rem(my + (2 * N_DEV - 1 - s), N_DEV)
        acc = rs_recv[s, :, :].astype(jnp.float32) + p_ref[
            pl.ds(c * blk, blk), :
        ].astype(jnp.float32)
        if s < N_DEV - 2:
            stage[s + 1, :, :] = acc.astype(jnp.bfloat16)
        else:
            own = lax.rem(my + 1, N_DEV)
            out_ref[pl.ds(own * blk, blk), :] = acc
            stage[N_DEV - 1, :, :] = acc.astype(jnp.bfloat16)

    for s in range(N_DEV - 1):
        src = stage.at[N_DEV - 1] if s == 0 else ag_recv.at[s - 1]
        rdma = pltpu.make_async_remote_copy(
            src_ref=src,
            dst_ref=ag_recv.at[s],
            send_sem=ag_send_sem.at[s],
            recv_sem=ag_recv_sem.at[s],
            device_id=(right,),
            device_id_type=pl.DeviceIdType.MESH,
        )
        rdma.start()
        rdma.wait()
        c = lax.rem(my + (N_DEV - s), N_DEV)
        out_ref[pl.ds(c * blk, blk), :] = ag_recv[s, :, :].astype(jnp.float32)


def _allreduce(p):
    m, n = p.shape
    blk = m // N_DEV
    return pl.pallas_call(
        _allreduce_body,
        out_shape=jax.ShapeDtypeStruct((m, n), jnp.float32),
        in_specs=[pl.BlockSpec(memory_space=pltpu.VMEM)],
        out_specs=pl.BlockSpec(memory_space=pltpu.VMEM),
        scratch_shapes=[
            pltpu.VMEM((N_DEV - 1, blk, n), jnp.bfloat16),
            pltpu.VMEM((N_DEV, blk, n), jnp.bfloat16),
            pltpu.VMEM((N_DEV - 1, blk, n), jnp.bfloat16),
            pltpu.SemaphoreType.DMA((N_DEV - 1,)),
            pltpu.SemaphoreType.DMA((N_DEV - 1,)),
            pltpu.SemaphoreType.DMA((N_DEV - 1,)),
            pltpu.SemaphoreType.DMA((N_DEV - 1,)),
        ],
        compiler_params=pltpu.CompilerParams(collective_id=0),
    )(p)


def kernel(x, W1, W2):
    xb = x.astype(jnp.bfloat16)
    w1b = W1.astype(jnp.bfloat16)
    w2b = W2.astype(jnp.bfloat16)
    p = _mlp(xb, w1b, w2b)
    return _allreduce(p)


# baseline (device time: 298290 ns/iter reference)
import jax
import jax.numpy as jnp
from jax import lax
from jax.experimental import pallas as pl
from jax.experimental.pallas import tpu as pltpu

N_DEV = 4


_MLP_ROW_BLK = 256


def _mlp_body(x_ref, w1_ref, w2_ref, out_ref):
    h = jnp.dot(x_ref[:, :], w1_ref[:, :], preferred_element_type=jnp.float32)
    h = jnp.maximum(h, 0.0).astype(jnp.bfloat16)
    p = jnp.dot(h, w2_ref[:, :], preferred_element_type=jnp.float32)
    out_ref[:, :] = p.astype(jnp.bfloat16)


def _mlp(x, w1, w2):
    m, k = x.shape
    n = w2.shape[1]
    return pl.pallas_call(
        _mlp_body,
        grid=(m // _MLP_ROW_BLK,),
        out_shape=jax.ShapeDtypeStruct((m, n), jnp.bfloat16),
        in_specs=[
            pl.BlockSpec((_MLP_ROW_BLK, k), lambda i: (i, 0)),
            pl.BlockSpec(memory_space=pltpu.VMEM),
            pl.BlockSpec(memory_space=pltpu.VMEM),
        ],
        out_specs=pl.BlockSpec((_MLP_ROW_BLK, n), lambda i: (i, 0)),
        compiler_params=pltpu.CompilerParams(
            vmem_limit_bytes=64 * 1024 * 1024,
        ),
    )(x, w1, w2)


def _allreduce_body(
    p_ref,
    out_ref,
    rs_recv,
    stage,
    ag_recv,
    rs_send_sem,
    rs_recv_sem,
    ag_send_sem,
    ag_recv_sem,
):
    my = lax.axis_index("i")
    left = lax.rem(my + N_DEV - 1, N_DEV)
    right = lax.rem(my + 1, N_DEV)
    blk = p_ref.shape[0] // N_DEV

    barrier = pltpu.get_barrier_semaphore()
    for nbr in (left, right):
        pl.semaphore_signal(
            barrier, inc=1, device_id=(nbr,), device_id_type=pl.DeviceIdType.MESH
        )
    pl.semaphore_wait(barrier, 2)

    stage[0, :, :] = p_ref[pl.ds(my * blk, blk), :]
    for s in range(N_DEV - 1):
        rdma = pltpu.make_async_remote_copy(
            src_ref=stage.at[s],
            dst_ref=rs_recv.at[s],
            send_sem=rs_send_sem.at[s],
            recv_sem=rs_recv_sem.at[s],
            device_id=(right,),
            device_id_type=pl.DeviceIdType.MESH,
        )
        rdma.start()
        rdma.wait()
        c = lax.rem(my + (2 * N_DEV - 1 - s), N_DEV)
        acc = rs_recv[s, :, :].astype(jnp.float32) + p_ref[
            pl.ds(c * blk, blk), :
        ].astype(jnp.float32)
        if s < N_DEV - 2:
            stage[s + 1, :, :] = acc.astype(jnp.bfloat16)
        else:
            own = lax.rem(my + 1, N_DEV)
            out_ref[pl.ds(own * blk, blk), :] = acc
            stage[N_DEV - 1, :, :] = acc.astype(jnp.bfloat16)

    for s in range(N_DEV - 1):
        src = stage.at[N_DEV - 1] if s == 0 else ag_recv.at[s - 1]
        rdma = pltpu.make_async_remote_copy(
            src_ref=src,
            dst_ref=ag_recv.at[s],
            send_sem=ag_send_sem.at[s],
            recv_sem=ag_recv_sem.at[s],
            device_id=(right,),
            device_id_type=pl.DeviceIdType.MESH,
        )
        rdma.start()
        rdma.wait()
        c = lax.rem(my + (N_DEV - s), N_DEV)
        out_ref[pl.ds(c * blk, blk), :] = ag_recv[s, :, :].astype(jnp.float32)


def _allreduce(p):
    m, n = p.shape
    blk = m // N_DEV
    return pl.pallas_call(
        _allreduce_body,
        out_shape=jax.ShapeDtypeStruct((m, n), jnp.float32),
        in_specs=[pl.BlockSpec(memory_space=pltpu.VMEM)],
        out_specs=pl.BlockSpec(memory_space=pltpu.VMEM),
        scratch_shapes=[
            pltpu.VMEM((N_DEV - 1, blk, n), jnp.bfloat16),
            pltpu.VMEM((N_DEV, blk, n), jnp.bfloat16),
            pltpu.VMEM((N_DEV - 1, blk, n), jnp.bfloat16),
            pltpu.SemaphoreType.DMA((N_DEV - 1,)),
            pltpu.SemaphoreType.DMA((N_DEV - 1,)),
            pltpu.SemaphoreType.DMA((N_DEV - 1,)),
            pltpu.SemaphoreType.DMA((N_DEV - 1,)),
        ],
        compiler_params=pltpu.CompilerParams(
            collective_id=0,
            vmem_limit_bytes=64 * 1024 * 1024,
        ),
    )(p)


def kernel(x, W1, W2):
    xb = x.astype(jnp.bfloat16)
    w1b = W1.astype(jnp.bfloat16)
    w2b = W2.astype(jnp.bfloat16)
    p = _mlp(xb, w1b, w2b)
    return _allreduce(p)


# device time: 231281 ns/iter; 1.2897x vs baseline; 1.2897x over previous
import jax
import jax.numpy as jnp
from jax import lax
from jax.experimental import pallas as pl
from jax.experimental.pallas import tpu as pltpu

N_DEV = 4


_MLP_ROW_BLK = 256


def _mlp_body(x_ref, w1_ref, w2_ref, out_ref):
    h = jnp.dot(x_ref[:, :], w1_ref[:, :], preferred_element_type=jnp.float32)
    h = jnp.maximum(h, 0.0).astype(jnp.bfloat16)
    p = jnp.dot(h, w2_ref[:, :], preferred_element_type=jnp.float32)
    out_ref[:, :] = p.astype(jnp.bfloat16)


def _mlp(x, w1, w2):
    m, k = x.shape
    n = w2.shape[1]
    return pl.pallas_call(
        _mlp_body,
        grid=(m // _MLP_ROW_BLK,),
        out_shape=jax.ShapeDtypeStruct((m, n), jnp.bfloat16),
        in_specs=[
            pl.BlockSpec((_MLP_ROW_BLK, k), lambda i: (i, 0)),
            pl.BlockSpec(memory_space=pltpu.VMEM),
            pl.BlockSpec(memory_space=pltpu.VMEM),
        ],
        out_specs=pl.BlockSpec((_MLP_ROW_BLK, n), lambda i: (i, 0)),
        compiler_params=pltpu.CompilerParams(
            vmem_limit_bytes=64 * 1024 * 1024,
        ),
    )(x, w1, w2)


def _allreduce_body(
    p_ref,
    out_ref,
    rs_recv,
    stage,
    ag_recv,
    rs_send_sem,
    rs_recv_sem,
    ag_send_sem,
    ag_recv_sem,
):
    my = lax.axis_index("i")
    left = lax.rem(my + N_DEV - 1, N_DEV)
    right = lax.rem(my + 1, N_DEV)
    blk = p_ref.shape[0] // N_DEV
    half = blk // 2

    barrier = pltpu.get_barrier_semaphore()
    for nbr in (left, right):
        pl.semaphore_signal(
            barrier, inc=1, device_id=(nbr,), device_id_type=pl.DeviceIdType.MESH
        )
    pl.semaphore_wait(barrier, 2)

    def row0(c):
        return c * blk

    def row1(c):
        return c * blk + half

    stage[0, 0, :, :] = p_ref[pl.ds(row0(my), half), :]
    stage[1, 0, :, :] = p_ref[pl.ds(row1(my), half), :]

    def hop(src_refs, dst, send_sem, recv_sem, s):
        rdmas = []
        for d, tgt in ((0, right), (1, left)):
            rdma = pltpu.make_async_remote_copy(
                src_ref=src_refs[d],
                dst_ref=dst.at[d, s],
                send_sem=send_sem.at[d, s],
                recv_sem=recv_sem.at[d, s],
                device_id=(tgt,),
                device_id_type=pl.DeviceIdType.MESH,
            )
            rdma.start()
            rdmas.append(rdma)
        for rdma in rdmas:
            rdma.wait()

    for s in range(N_DEV - 1):
        hop((stage.at[0, s], stage.at[1, s]), rs_recv, rs_send_sem, rs_recv_sem, s)
        c_r = lax.rem(my + (2 * N_DEV - 1 - s), N_DEV)
        c_l = lax.rem(my + s + 1, N_DEV)
        acc_r = rs_recv[0, s, :, :].astype(jnp.float32) + p_ref[
            pl.ds(row0(c_r), half), :
        ].astype(jnp.float32)
        acc_l = rs_recv[1, s, :, :].astype(jnp.float32) + p_ref[
            pl.ds(row1(c_l), half), :
        ].astype(jnp.float32)
        if s < N_DEV - 2:
            stage[0, s + 1, :, :] = acc_r.astype(jnp.bfloat16)
            stage[1, s + 1, :, :] = acc_l.astype(jnp.bfloat16)
        else:
            out_ref[pl.ds(row0(c_r), half), :] = acc_r
            out_ref[pl.ds(row1(c_l), half), :] = acc_l
            stage[0, N_DEV - 1, :, :] = acc_r.astype(jnp.bfloat16)
            stage[1, N_DEV - 1, :, :] = acc_l.astype(jnp.bfloat16)

    for s in range(N_DEV - 1):
        if s == 0:
            srcs = (stage.at[0, N_DEV - 1], stage.at[1, N_DEV - 1])
        else:
            srcs = (ag_recv.at[0, s - 1], ag_recv.at[1, s - 1])
        hop(srcs, ag_recv, ag_send_sem, ag_recv_sem, s)
        c_r = lax.rem(my + (N_DEV - s), N_DEV)
        c_l = lax.rem(my + s, N_DEV)
        out_ref[pl.ds(row0(c_r), half), :] = ag_recv[0, s, :, :].astype(jnp.float32)
        out_ref[pl.ds(row1(c_l), half), :] = ag_recv[1, s, :, :].astype(jnp.float32)


def _allreduce(p):
    m, n = p.shape
    half = m // N_DEV // 2
    return pl.pallas_call(
        _allreduce_body,
        out_shape=jax.ShapeDtypeStruct((m, n), jnp.float32),
        in_specs=[pl.BlockSpec(memory_space=pltpu.VMEM)],
        out_specs=pl.BlockSpec(memory_space=pltpu.VMEM),
        scratch_shapes=[
            pltpu.VMEM((2, N_DEV - 1, half, n), jnp.bfloat16),
            pltpu.VMEM((2, N_DEV, half, n), jnp.bfloat16),
            pltpu.VMEM((2, N_DEV - 1, half, n), jnp.bfloat16),
            pltpu.SemaphoreType.DMA((2, N_DEV - 1)),
            pltpu.SemaphoreType.DMA((2, N_DEV - 1)),
            pltpu.SemaphoreType.DMA((2, N_DEV - 1)),
            pltpu.SemaphoreType.DMA((2, N_DEV - 1)),
        ],
        compiler_params=pltpu.CompilerParams(
            collective_id=0,
            vmem_limit_bytes=64 * 1024 * 1024,
        ),
    )(p)


def kernel(x, W1, W2):
    xb = x.astype(jnp.bfloat16)
    w1b = W1.astype(jnp.bfloat16)
    w2b = W2.astype(jnp.bfloat16)
    p = _mlp(xb, w1b, w2b)
    return _allreduce(p)


# device time: 219636 ns/iter; 1.3581x vs baseline; 1.0530x over previous
import jax

jax.config.update("jax_compilation_cache_dir", "/tmp/jax_comp_cache")
jax.config.update("jax_persistent_cache_min_compile_time_secs", 1)

import jax.numpy as jnp
from jax import lax
from jax.experimental import pallas as pl
from jax.experimental.pallas import tpu as pltpu

N_DEV = 4


_MLP_ROW_BLK = 256


def _mlp_body(x_ref, w1_ref, w2_ref, out_ref):
    h = jnp.dot(x_ref[:, :], w1_ref[:, :], preferred_element_type=jnp.float32)
    h = jnp.maximum(h, 0.0).astype(jnp.bfloat16)
    p = jnp.dot(h, w2_ref[:, :], preferred_element_type=jnp.float32)
    out_ref[:, :] = p.astype(jnp.bfloat16)


def _mlp(x, w1, w2):
    m, k = x.shape
    n = w2.shape[1]
    return pl.pallas_call(
        _mlp_body,
        grid=(m // _MLP_ROW_BLK,),
        out_shape=jax.ShapeDtypeStruct((m, n), jnp.bfloat16),
        in_specs=[
            pl.BlockSpec((_MLP_ROW_BLK, k), lambda i: (i, 0)),
            pl.BlockSpec(memory_space=pltpu.VMEM),
            pl.BlockSpec(memory_space=pltpu.VMEM),
        ],
        out_specs=pl.BlockSpec((_MLP_ROW_BLK, n), lambda i: (i, 0)),
        compiler_params=pltpu.CompilerParams(
            vmem_limit_bytes=64 * 1024 * 1024,
        ),
    )(x, w1, w2)


def _allreduce_body(
    p_ref,
    out_ref,
    rs_recv,
    stage,
    ag_recv,
    rs_send_sem,
    rs_recv_sem,
    ag_send_sem,
    ag_recv_sem,
):
    my = lax.axis_index("i")
    left = lax.rem(my + N_DEV - 1, N_DEV)
    right = lax.rem(my + 1, N_DEV)
    blk = p_ref.shape[0] // N_DEV
    half = blk // 2

    barrier = pltpu.get_barrier_semaphore()
    for nbr in (left, right):
        pl.semaphore_signal(
            barrier, inc=1, device_id=(nbr,), device_id_type=pl.DeviceIdType.MESH
        )
    pl.semaphore_wait(barrier, 2)

    def row0(c):
        return c * blk

    def row1(c):
        return c * blk + half

    stage[0, 0, :, :] = p_ref[pl.ds(row0(my), half), :]
    stage[1, 0, :, :] = p_ref[pl.ds(row1(my), half), :]

    def hop(src_refs, dst, send_sem, recv_sem, s):
        rdmas = []
        for d, tgt in ((0, right), (1, left)):
            rdma = pltpu.make_async_remote_copy(
                src_ref=src_refs[d],
                dst_ref=dst.at[d, s],
                send_sem=send_sem.at[d, s],
                recv_sem=recv_sem.at[d, s],
                device_id=(tgt,),
                device_id_type=pl.DeviceIdType.MESH,
            )
            rdma.start()
            rdmas.append(rdma)
        for rdma in rdmas:
            rdma.wait()

    for s in range(N_DEV - 1):
        hop((stage.at[0, s], stage.at[1, s]), rs_recv, rs_send_sem, rs_recv_sem, s)
        c_r = lax.rem(my + (2 * N_DEV - 1 - s), N_DEV)
        c_l = lax.rem(my + s + 1, N_DEV)
        acc_r = rs_recv[0, s, :, :].astype(jnp.float32) + p_ref[
            pl.ds(row0(c_r), half), :
        ].astype(jnp.float32)
        acc_l = rs_recv[1, s, :, :].astype(jnp.float32) + p_ref[
            pl.ds(row1(c_l), half), :
        ].astype(jnp.float32)
        if s < N_DEV - 2:
            stage[0, s + 1, :, :] = acc_r.astype(jnp.bfloat16)
            stage[1, s + 1, :, :] = acc_l.astype(jnp.bfloat16)
        else:
            out_ref[pl.ds(row0(c_r), half), :] = acc_r
            out_ref[pl.ds(row1(c_l), half), :] = acc_l
            stage[0, N_DEV - 1, :, :] = acc_r.astype(jnp.bfloat16)
            stage[1, N_DEV - 1, :, :] = acc_l.astype(jnp.bfloat16)

    for s in range(N_DEV - 1):
        if s == 0:
            srcs = (stage.at[0, N_DEV - 1], stage.at[1, N_DEV - 1])
        else:
            srcs = (ag_recv.at[0, s - 1], ag_recv.at[1, s - 1])
        hop(srcs, ag_recv, ag_send_sem, ag_recv_sem, s)
        c_r = lax.rem(my + (N_DEV - s), N_DEV)
        c_l = lax.rem(my + s, N_DEV)
        out_ref[pl.ds(row0(c_r), half), :] = ag_recv[0, s, :, :].astype(jnp.float32)
        out_ref[pl.ds(row1(c_l), half), :] = ag_recv[1, s, :, :].astype(jnp.float32)


def _allreduce(p):
    m, n = p.shape
    half = m // N_DEV // 2
    return pl.pallas_call(
        _allreduce_body,
        out_shape=jax.ShapeDtypeStruct((m, n), jnp.float32),
        in_specs=[pl.BlockSpec(memory_space=pltpu.VMEM)],
        out_specs=pl.BlockSpec(memory_space=pltpu.VMEM),
        scratch_shapes=[
            pltpu.VMEM((2, N_DEV - 1, half, n), jnp.bfloat16),
            pltpu.VMEM((2, N_DEV, half, n), jnp.bfloat16),
            pltpu.VMEM((2, N_DEV - 1, half, n), jnp.bfloat16),
            pltpu.SemaphoreType.DMA((2, N_DEV - 1)),
            pltpu.SemaphoreType.DMA((2, N_DEV - 1)),
            pltpu.SemaphoreType.DMA((2, N_DEV - 1)),
            pltpu.SemaphoreType.DMA((2, N_DEV - 1)),
        ],
        compiler_params=pltpu.CompilerParams(
            collective_id=0,
            vmem_limit_bytes=64 * 1024 * 1024,
        ),
    )(p)


def _fused_body(
    x_hbm,
    w1_ref,
    w2_ref,
    out_ref,
    xland,
    seed,
    stage,
    rs_recv,
    ag_recv,
    x_sem,
    out_sem,
    rs_send_sem,
    rs_recv_sem,
    ag_send_sem,
    ag_recv_sem,
):
    my = lax.axis_index("i")
    left = lax.rem(my + N_DEV - 1, N_DEV)
    right = lax.rem(my + 1, N_DEV)
    blk = out_ref.shape[0] // N_DEV
    half = blk // 2

    def load_half(c, d):
        cp = pltpu.make_async_copy(
            x_hbm.at[pl.ds(c * blk + d * half, half), :], xland.at[d], x_sem.at[d]
        )
        cp.start()
        return cp

    def compute(d):
        xb = xland[d, :, :].astype(jnp.bfloat16)
        h = jnp.dot(xb, w1_ref[:, :], preferred_element_type=jnp.float32)
        h = jnp.maximum(h, 0.0).astype(jnp.bfloat16)
        return jnp.dot(h, w2_ref[:, :], preferred_element_type=jnp.float32)

    def start_rs(s, src_t, src_b):
        rt = pltpu.make_async_remote_copy(
            src_ref=src_t,
            dst_ref=rs_recv.at[0, s],
            send_sem=rs_send_sem.at[0, s],
            recv_sem=rs_recv_sem.at[0, s],
            device_id=(right,),
            device_id_type=pl.DeviceIdType.MESH,
        )
        rb = pltpu.make_async_remote_copy(
            src_ref=src_b,
            dst_ref=rs_recv.at[1, s],
            send_sem=rs_send_sem.at[1, s],
            recv_sem=rs_recv_sem.at[1, s],
            device_id=(left,),
            device_id_type=pl.DeviceIdType.MESH,
        )
        rt.start()
        rb.start()
        return rt, rb

    cp0 = load_half(my, 0)
    cp1 = load_half(my, 1)
    cp0.wait()
    seed[0, :, :] = compute(0).astype(jnp.bfloat16)
    cp1.wait()
    seed[1, :, :] = compute(1).astype(jnp.bfloat16)

    barrier = pltpu.get_barrier_semaphore()
    for nbr in (left, right):
        pl.semaphore_signal(
            barrier, inc=1, device_id=(nbr,), device_id_type=pl.DeviceIdType.MESH
        )
    pl.semaphore_wait(barrier, 2)

    pend = start_rs(0, seed.at[0], seed.at[1])
    for s in range(N_DEV - 1):
        c_t = lax.rem(my + (2 * N_DEV - 1 - s), N_DEV)
        c_b = lax.rem(my + s + 1, N_DEV)
        pend[0].wait()
        pend[1].wait()
        cp0 = load_half(c_t, 0)
        cp1 = load_half(c_b, 1)
        cp0.wait()
        p_t = compute(0)
        cp1.wait()
        p_b = compute(1)
        acc_t = rs_recv[0, s, :, :].astype(jnp.float32) + p_t
        acc_b = rs_recv[1, s, :, :].astype(jnp.float32) + p_b
        if s < N_DEV - 2:
            stage[0, s, :, :] = acc_t.astype(jnp.bfloat16)
            stage[1, s, :, :] = acc_b.astype(jnp.bfloat16)
            pend = start_rs(s + 1, stage.at[0, s], stage.at[1, s])
        else:
            stage[0, N_DEV - 2, :, :] = acc_t.astype(jnp.bfloat16)
            stage[1, N_DEV - 2, :, :] = acc_b.astype(jnp.bfloat16)

    out_dmas = []

    def store_out(src, c, d, j):
        cp = pltpu.make_async_copy(
            src, out_ref.at[pl.ds(c * blk + d * half, half), :], out_sem.at[d, j]
        )
        cp.start()
        out_dmas.append(cp)

    store_out(stage.at[0, N_DEV - 2], lax.rem(my + 1, N_DEV), 0, 0)
    store_out(stage.at[1, N_DEV - 2], left, 1, 0)

    for s in range(N_DEV - 1):
        srcs = (
            (stage.at[0, N_DEV - 2], stage.at[1, N_DEV - 2])
            if s == 0
            else (ag_recv.at[0, s - 1], ag_recv.at[1, s - 1])
        )
        rt = pltpu.make_async_remote_copy(
            src_ref=srcs[0],
            dst_ref=ag_recv.at[0, s],
            send_sem=ag_send_sem.at[0, s],
            recv_sem=ag_recv_sem.at[0, s],
            device_id=(right,),
            device_id_type=pl.DeviceIdType.MESH,
        )
        rb = pltpu.make_async_remote_copy(
            src_ref=srcs[1],
            dst_ref=ag_recv.at[1, s],
            send_sem=ag_send_sem.at[1, s],
            recv_sem=ag_recv_sem.at[1, s],
            device_id=(left,),
            device_id_type=pl.DeviceIdType.MESH,
        )
        rt.start()
        rb.start()
        rt.wait()
        rb.wait()
        c_t = lax.rem(my + N_DEV - s, N_DEV)
        c_b = lax.rem(my + s, N_DEV)
        store_out(ag_recv.at[0, s], c_t, 0, s + 1)
        store_out(ag_recv.at[1, s], c_b, 1, s + 1)

    for cp in out_dmas:
        cp.wait()


def _fused(x, w1, w2):
    m, k = x.shape
    n = w2.shape[1]
    half = m // N_DEV // 2
    return pl.pallas_call(
        _fused_body,
        out_shape=jax.ShapeDtypeStruct((m, n), jnp.bfloat16),
        in_specs=[
            pl.BlockSpec(memory_space=pl.ANY),
            pl.BlockSpec(memory_space=pltpu.VMEM),
            pl.BlockSpec(memory_space=pltpu.VMEM),
        ],
        out_specs=pl.BlockSpec(memory_space=pl.ANY),
        scratch_shapes=[
            pltpu.VMEM((2, half, k), jnp.float32),
            pltpu.VMEM((2, half, n), jnp.bfloat16),
            pltpu.VMEM((2, N_DEV - 1, half, n), jnp.bfloat16),
            pltpu.VMEM((2, N_DEV - 1, half, n), jnp.bfloat16),
            pltpu.VMEM((2, N_DEV - 1, half, n), jnp.bfloat16),
            pltpu.SemaphoreType.DMA((2,)),
            pltpu.SemaphoreType.DMA((2, N_DEV)),
            pltpu.SemaphoreType.DMA((2, N_DEV - 1)),
            pltpu.SemaphoreType.DMA((2, N_DEV - 1)),
            pltpu.SemaphoreType.DMA((2, N_DEV - 1)),
            pltpu.SemaphoreType.DMA((2, N_DEV - 1)),
        ],
        compiler_params=pltpu.CompilerParams(
            collective_id=0,
            vmem_limit_bytes=64 * 1024 * 1024,
        ),
    )(x, w1, w2)


def kernel(x, W1, W2):
    w1b = W1.astype(jnp.bfloat16)
    w2b = W2.astype(jnp.bfloat16)
    return _fused(x, w1b, w2b).astype(jnp.float32)


# device time: 179966 ns/iter; 1.6575x vs baseline; 1.2204x over previous
import jax

jax.config.update("jax_compilation_cache_dir", "/tmp/jax_comp_cache")
jax.config.update("jax_persistent_cache_min_compile_time_secs", 1)

import jax.numpy as jnp
from jax import lax
from jax.experimental import pallas as pl
from jax.experimental.pallas import tpu as pltpu

N_DEV = 4


_MLP_ROW_BLK = 256


def _mlp_body(x_ref, w1_ref, w2_ref, out_ref):
    h = jnp.dot(x_ref[:, :], w1_ref[:, :], preferred_element_type=jnp.float32)
    h = jnp.maximum(h, 0.0).astype(jnp.bfloat16)
    p = jnp.dot(h, w2_ref[:, :], preferred_element_type=jnp.float32)
    out_ref[:, :] = p.astype(jnp.bfloat16)


def _mlp(x, w1, w2):
    m, k = x.shape
    n = w2.shape[1]
    return pl.pallas_call(
        _mlp_body,
        grid=(m // _MLP_ROW_BLK,),
        out_shape=jax.ShapeDtypeStruct((m, n), jnp.bfloat16),
        in_specs=[
            pl.BlockSpec((_MLP_ROW_BLK, k), lambda i: (i, 0)),
            pl.BlockSpec(memory_space=pltpu.VMEM),
            pl.BlockSpec(memory_space=pltpu.VMEM),
        ],
        out_specs=pl.BlockSpec((_MLP_ROW_BLK, n), lambda i: (i, 0)),
        compiler_params=pltpu.CompilerParams(
            vmem_limit_bytes=64 * 1024 * 1024,
        ),
    )(x, w1, w2)


def _allreduce_body(
    p_ref,
    out_ref,
    rs_recv,
    stage,
    ag_recv,
    rs_send_sem,
    rs_recv_sem,
    ag_send_sem,
    ag_recv_sem,
):
    my = lax.axis_index("i")
    left = lax.rem(my + N_DEV - 1, N_DEV)
    right = lax.rem(my + 1, N_DEV)
    blk = p_ref.shape[0] // N_DEV
    half = blk // 2

    barrier = pltpu.get_barrier_semaphore()
    for nbr in (left, right):
        pl.semaphore_signal(
            barrier, inc=1, device_id=(nbr,), device_id_type=pl.DeviceIdType.MESH
        )
    pl.semaphore_wait(barrier, 2)

    def row0(c):
        return c * blk

    def row1(c):
        return c * blk + half

    stage[0, 0, :, :] = p_ref[pl.ds(row0(my), half), :]
    stage[1, 0, :, :] = p_ref[pl.ds(row1(my), half), :]

    def hop(src_refs, dst, send_sem, recv_sem, s):
        rdmas = []
        for d, tgt in ((0, right), (1, left)):
            rdma = pltpu.make_async_remote_copy(
                src_ref=src_refs[d],
                dst_ref=dst.at[d, s],
                send_sem=send_sem.at[d, s],
                recv_sem=recv_sem.at[d, s],
                device_id=(tgt,),
                device_id_type=pl.DeviceIdType.MESH,
            )
            rdma.start()
            rdmas.append(rdma)
        for rdma in rdmas:
            rdma.wait()

    for s in range(N_DEV - 1):
        hop((stage.at[0, s], stage.at[1, s]), rs_recv, rs_send_sem, rs_recv_sem, s)
        c_r = lax.rem(my + (2 * N_DEV - 1 - s), N_DEV)
        c_l = lax.rem(my + s + 1, N_DEV)
        acc_r = rs_recv[0, s, :, :].astype(jnp.float32) + p_ref[
            pl.ds(row0(c_r), half), :
        ].astype(jnp.float32)
        acc_l = rs_recv[1, s, :, :].astype(jnp.float32) + p_ref[
            pl.ds(row1(c_l), half), :
        ].astype(jnp.float32)
        if s < N_DEV - 2:
            stage[0, s + 1, :, :] = acc_r.astype(jnp.bfloat16)
            stage[1, s + 1, :, :] = acc_l.astype(jnp.bfloat16)
        else:
            out_ref[pl.ds(row0(c_r), half), :] = acc_r
            out_ref[pl.ds(row1(c_l), half), :] = acc_l
            stage[0, N_DEV - 1, :, :] = acc_r.astype(jnp.bfloat16)
            stage[1, N_DEV - 1, :, :] = acc_l.astype(jnp.bfloat16)

    for s in range(N_DEV - 1):
        if s == 0:
            srcs = (stage.at[0, N_DEV - 1], stage.at[1, N_DEV - 1])
        else:
            srcs = (ag_recv.at[0, s - 1], ag_recv.at[1, s - 1])
        hop(srcs, ag_recv, ag_send_sem, ag_recv_sem, s)
        c_r = lax.rem(my + (N_DEV - s), N_DEV)
        c_l = lax.rem(my + s, N_DEV)
        out_ref[pl.ds(row0(c_r), half), :] = ag_recv[0, s, :, :].astype(jnp.float32)
        out_ref[pl.ds(row1(c_l), half), :] = ag_recv[1, s, :, :].astype(jnp.float32)


def _allreduce(p):
    m, n = p.shape
    half = m // N_DEV // 2
    return pl.pallas_call(
        _allreduce_body,
        out_shape=jax.ShapeDtypeStruct((m, n), jnp.float32),
        in_specs=[pl.BlockSpec(memory_space=pltpu.VMEM)],
        out_specs=pl.BlockSpec(memory_space=pltpu.VMEM),
        scratch_shapes=[
            pltpu.VMEM((2, N_DEV - 1, half, n), jnp.bfloat16),
            pltpu.VMEM((2, N_DEV, half, n), jnp.bfloat16),
            pltpu.VMEM((2, N_DEV - 1, half, n), jnp.bfloat16),
            pltpu.SemaphoreType.DMA((2, N_DEV - 1)),
            pltpu.SemaphoreType.DMA((2, N_DEV - 1)),
            pltpu.SemaphoreType.DMA((2, N_DEV - 1)),
            pltpu.SemaphoreType.DMA((2, N_DEV - 1)),
        ],
        compiler_params=pltpu.CompilerParams(
            collective_id=0,
            vmem_limit_bytes=64 * 1024 * 1024,
        ),
    )(p)


def _fused_body(
    x_hbm,
    w1_ref,
    w2_ref,
    out_ref,
    xland,
    seed,
    stage,
    rs_recv,
    ag_recv,
    x_sem,
    out_sem,
    rs_send_sem,
    rs_recv_sem,
    ag_send_sem,
    ag_recv_sem,
):
    my = lax.axis_index("i")
    left = lax.rem(my + N_DEV - 1, N_DEV)
    right = lax.rem(my + 1, N_DEV)
    blk = out_ref.shape[0] // N_DEV
    half = blk // 2

    def load_half(c, d):
        cp = pltpu.make_async_copy(
            x_hbm.at[pl.ds(c * blk + d * half, half), :], xland.at[d], x_sem.at[d]
        )
        cp.start()
        return cp

    def compute(d):
        xb = xland[d, :, :].astype(jnp.bfloat16)
        h = jnp.dot(xb, w1_ref[:, :], preferred_element_type=jnp.float32)
        h = jnp.maximum(h, 0.0).astype(jnp.bfloat16)
        return jnp.dot(h, w2_ref[:, :], preferred_element_type=jnp.float32)

    def start_rs(s, src_t, src_b):
        rt = pltpu.make_async_remote_copy(
            src_ref=src_t,
            dst_ref=rs_recv.at[0, s],
            send_sem=rs_send_sem.at[0, s],
            recv_sem=rs_recv_sem.at[0, s],
            device_id=(right,),
            device_id_type=pl.DeviceIdType.MESH,
        )
        rb = pltpu.make_async_remote_copy(
            src_ref=src_b,
            dst_ref=rs_recv.at[1, s],
            send_sem=rs_send_sem.at[1, s],
            recv_sem=rs_recv_sem.at[1, s],
            device_id=(left,),
            device_id_type=pl.DeviceIdType.MESH,
        )
        rt.start()
        rb.start()
        return rt, rb

    cp0 = load_half(my, 0)
    cp1 = load_half(my, 1)
    cp0.wait()
    seed[0, :, :] = compute(0).astype(jnp.bfloat16)
    cp1.wait()
    seed[1, :, :] = compute(1).astype(jnp.bfloat16)

    barrier = pltpu.get_barrier_semaphore()
    for nbr in (left, right):
        pl.semaphore_signal(
            barrier, inc=1, device_id=(nbr,), device_id_type=pl.DeviceIdType.MESH
        )
    pl.semaphore_wait(barrier, 2)

    pend = start_rs(0, seed.at[0], seed.at[1])
    for s in range(N_DEV - 1):
        c_t = lax.rem(my + (2 * N_DEV - 1 - s), N_DEV)
        c_b = lax.rem(my + s + 1, N_DEV)
        cp0 = load_half(c_t, 0)
        cp1 = load_half(c_b, 1)
        cp0.wait()
        p_t = compute(0)
        cp1.wait()
        p_b = compute(1)
        pend[0].wait()
        pend[1].wait()
        acc_t = rs_recv[0, s, :, :].astype(jnp.float32) + p_t
        acc_b = rs_recv[1, s, :, :].astype(jnp.float32) + p_b
        if s < N_DEV - 2:
            stage[0, s, :, :] = acc_t.astype(jnp.bfloat16)
            stage[1, s, :, :] = acc_b.astype(jnp.bfloat16)
            pend = start_rs(s + 1, stage.at[0, s], stage.at[1, s])
        else:
            stage[0, N_DEV - 2, :, :] = acc_t.astype(jnp.bfloat16)
            stage[1, N_DEV - 2, :, :] = acc_b.astype(jnp.bfloat16)

    out_dmas = []

    def store_out(src, c, d, j):
        cp = pltpu.make_async_copy(
            src, out_ref.at[pl.ds(c * blk + d * half, half), :], out_sem.at[d, j]
        )
        cp.start()
        out_dmas.append(cp)

    store_out(stage.at[0, N_DEV - 2], lax.rem(my + 1, N_DEV), 0, 0)
    store_out(stage.at[1, N_DEV - 2], left, 1, 0)

    for s in range(N_DEV - 1):
        srcs = (
            (stage.at[0, N_DEV - 2], stage.at[1, N_DEV - 2])
            if s == 0
            else (ag_recv.at[0, s - 1], ag_recv.at[1, s - 1])
        )
        rt = pltpu.make_async_remote_copy(
            src_ref=srcs[0],
            dst_ref=ag_recv.at[0, s],
            send_sem=ag_send_sem.at[0, s],
            recv_sem=ag_recv_sem.at[0, s],
            device_id=(right,),
            device_id_type=pl.DeviceIdType.MESH,
        )
        rb = pltpu.make_async_remote_copy(
            src_ref=srcs[1],
            dst_ref=ag_recv.at[1, s],
            send_sem=ag_send_sem.at[1, s],
            recv_sem=ag_recv_sem.at[1, s],
            device_id=(left,),
            device_id_type=pl.DeviceIdType.MESH,
        )
        rt.start()
        rb.start()
        rt.wait()
        rb.wait()
        c_t = lax.rem(my + N_DEV - s, N_DEV)
        c_b = lax.rem(my + s, N_DEV)
        store_out(ag_recv.at[0, s], c_t, 0, s + 1)
        store_out(ag_recv.at[1, s], c_b, 1, s + 1)

    for cp in out_dmas:
        cp.wait()


def _fused(x, w1, w2):
    m, k = x.shape
    n = w2.shape[1]
    half = m // N_DEV // 2
    return pl.pallas_call(
        _fused_body,
        out_shape=jax.ShapeDtypeStruct((m, n), jnp.bfloat16),
        in_specs=[
            pl.BlockSpec(memory_space=pl.ANY),
            pl.BlockSpec(memory_space=pltpu.VMEM),
            pl.BlockSpec(memory_space=pltpu.VMEM),
        ],
        out_specs=pl.BlockSpec(memory_space=pl.ANY),
        scratch_shapes=[
            pltpu.VMEM((2, half, k), jnp.float32),
            pltpu.VMEM((2, half, n), jnp.bfloat16),
            pltpu.VMEM((2, N_DEV - 1, half, n), jnp.bfloat16),
            pltpu.VMEM((2, N_DEV - 1, half, n), jnp.bfloat16),
            pltpu.VMEM((2, N_DEV - 1, half, n), jnp.bfloat16),
            pltpu.SemaphoreType.DMA((2,)),
            pltpu.SemaphoreType.DMA((2, N_DEV)),
            pltpu.SemaphoreType.DMA((2, N_DEV - 1)),
            pltpu.SemaphoreType.DMA((2, N_DEV - 1)),
            pltpu.SemaphoreType.DMA((2, N_DEV - 1)),
            pltpu.SemaphoreType.DMA((2, N_DEV - 1)),
        ],
        compiler_params=pltpu.CompilerParams(
            collective_id=0,
            vmem_limit_bytes=64 * 1024 * 1024,
        ),
    )(x, w1, w2)


def kernel(x, W1, W2):
    w1b = W1.astype(jnp.bfloat16)
    w2b = W2.astype(jnp.bfloat16)
    return _fused(x, w1b, w2b).astype(jnp.float32)


# device time: 179939 ns/iter; 1.6577x vs baseline; 1.0002x over previous
import jax

jax.config.update("jax_compilation_cache_dir", "/tmp/jax_comp_cache")
jax.config.update("jax_persistent_cache_min_compile_time_secs", 1)

import jax.numpy as jnp
from jax import lax
from jax.experimental import pallas as pl
from jax.experimental.pallas import tpu as pltpu

N_DEV = 4


_MLP_ROW_BLK = 256


def _mlp_body(x_ref, w1_ref, w2_ref, out_ref):
    h = jnp.dot(x_ref[:, :], w1_ref[:, :], preferred_element_type=jnp.float32)
    h = jnp.maximum(h, 0.0).astype(jnp.bfloat16)
    p = jnp.dot(h, w2_ref[:, :], preferred_element_type=jnp.float32)
    out_ref[:, :] = p.astype(jnp.bfloat16)


def _mlp(x, w1, w2):
    m, k = x.shape
    n = w2.shape[1]
    return pl.pallas_call(
        _mlp_body,
        grid=(m // _MLP_ROW_BLK,),
        out_shape=jax.ShapeDtypeStruct((m, n), jnp.bfloat16),
        in_specs=[
            pl.BlockSpec((_MLP_ROW_BLK, k), lambda i: (i, 0)),
            pl.BlockSpec(memory_space=pltpu.VMEM),
            pl.BlockSpec(memory_space=pltpu.VMEM),
        ],
        out_specs=pl.BlockSpec((_MLP_ROW_BLK, n), lambda i: (i, 0)),
        compiler_params=pltpu.CompilerParams(
            vmem_limit_bytes=64 * 1024 * 1024,
        ),
    )(x, w1, w2)


def _allreduce_body(
    p_ref,
    out_ref,
    rs_recv,
    stage,
    ag_recv,
    rs_send_sem,
    rs_recv_sem,
    ag_send_sem,
    ag_recv_sem,
):
    my = lax.axis_index("i")
    left = lax.rem(my + N_DEV - 1, N_DEV)
    right = lax.rem(my + 1, N_DEV)
    blk = p_ref.shape[0] // N_DEV
    half = blk // 2

    barrier = pltpu.get_barrier_semaphore()
    for nbr in (left, right):
        pl.semaphore_signal(
            barrier, inc=1, device_id=(nbr,), device_id_type=pl.DeviceIdType.MESH
        )
    pl.semaphore_wait(barrier, 2)

    def row0(c):
        return c * blk

    def row1(c):
        return c * blk + half

    stage[0, 0, :, :] = p_ref[pl.ds(row0(my), half), :]
    stage[1, 0, :, :] = p_ref[pl.ds(row1(my), half), :]

    def hop(src_refs, dst, send_sem, recv_sem, s):
        rdmas = []
        for d, tgt in ((0, right), (1, left)):
            rdma = pltpu.make_async_remote_copy(
                src_ref=src_refs[d],
                dst_ref=dst.at[d, s],
                send_sem=send_sem.at[d, s],
                recv_sem=recv_sem.at[d, s],
                device_id=(tgt,),
                device_id_type=pl.DeviceIdType.MESH,
            )
            rdma.start()
            rdmas.append(rdma)
        for rdma in rdmas:
            rdma.wait()

    for s in range(N_DEV - 1):
        hop((stage.at[0, s], stage.at[1, s]), rs_recv, rs_send_sem, rs_recv_sem, s)
        c_r = lax.rem(my + (2 * N_DEV - 1 - s), N_DEV)
        c_l = lax.rem(my + s + 1, N_DEV)
        acc_r = rs_recv[0, s, :, :].astype(jnp.float32) + p_ref[
            pl.ds(row0(c_r), half), :
        ].astype(jnp.float32)
        acc_l = rs_recv[1, s, :, :].astype(jnp.float32) + p_ref[
            pl.ds(row1(c_l), half), :
        ].astype(jnp.float32)
        if s < N_DEV - 2:
            stage[0, s + 1, :, :] = acc_r.astype(jnp.bfloat16)
            stage[1, s + 1, :, :] = acc_l.astype(jnp.bfloat16)
        else:
            out_ref[pl.ds(row0(c_r), half), :] = acc_r
            out_ref[pl.ds(row1(c_l), half), :] = acc_l
            stage[0, N_DEV - 1, :, :] = acc_r.astype(jnp.bfloat16)
            stage[1, N_DEV - 1, :, :] = acc_l.astype(jnp.bfloat16)

    for s in range(N_DEV - 1):
        if s == 0:
            srcs = (stage.at[0, N_DEV - 1], stage.at[1, N_DEV - 1])
        else:
            srcs = (ag_recv.at[0, s - 1], ag_recv.at[1, s - 1])
        hop(srcs, ag_recv, ag_send_sem, ag_recv_sem, s)
        c_r = lax.rem(my + (N_DEV - s), N_DEV)
        c_l = lax.rem(my + s, N_DEV)
        out_ref[pl.ds(row0(c_r), half), :] = ag_recv[0, s, :, :].astype(jnp.float32)
        out_ref[pl.ds(row1(c_l), half), :] = ag_recv[1, s, :, :].astype(jnp.float32)


def _allreduce(p):
    m, n = p.shape
    half = m // N_DEV // 2
    return pl.pallas_call(
        _allreduce_body,
        out_shape=jax.ShapeDtypeStruct((m, n), jnp.float32),
        in_specs=[pl.BlockSpec(memory_space=pltpu.VMEM)],
        out_specs=pl.BlockSpec(memory_space=pltpu.VMEM),
        scratch_shapes=[
            pltpu.VMEM((2, N_DEV - 1, half, n), jnp.bfloat16),
            pltpu.VMEM((2, N_DEV, half, n), jnp.bfloat16),
            pltpu.VMEM((2, N_DEV - 1, half, n), jnp.bfloat16),
            pltpu.SemaphoreType.DMA((2, N_DEV - 1)),
            pltpu.SemaphoreType.DMA((2, N_DEV - 1)),
            pltpu.SemaphoreType.DMA((2, N_DEV - 1)),
            pltpu.SemaphoreType.DMA((2, N_DEV - 1)),
        ],
        compiler_params=pltpu.CompilerParams(
            collective_id=0,
            vmem_limit_bytes=64 * 1024 * 1024,
        ),
    )(p)


def _fused_body(
    x_hbm,
    w1_ref,
    w2_ref,
    out_ref,
    xland,
    seed,
    stage,
    rs_recv,
    ag_recv,
    x_sem,
    out_sem,
    rs_send_sem,
    rs_recv_sem,
    ag_send_sem,
    ag_recv_sem,
):
    my = lax.axis_index("i")
    left = lax.rem(my + N_DEV - 1, N_DEV)
    right = lax.rem(my + 1, N_DEV)
    blk = out_ref.shape[0] // N_DEV
    half = blk // 2

    def load_half(c, d):
        cp = pltpu.make_async_copy(
            x_hbm.at[pl.ds(c * blk + d * half, half), :], xland.at[d], x_sem.at[d]
        )
        cp.start()
        return cp

    def compute(d):
        xb = xland[d, :, :].astype(jnp.bfloat16)
        h = jnp.dot(xb, w1_ref[:, :], preferred_element_type=jnp.float32)
        h = jnp.maximum(h, 0.0).astype(jnp.bfloat16)
        return jnp.dot(h, w2_ref[:, :], preferred_element_type=jnp.float32)

    def start_dir(d, tgt, s, src):
        rdma = pltpu.make_async_remote_copy(
            src_ref=src,
            dst_ref=rs_recv.at[d, s],
            send_sem=rs_send_sem.at[d, s],
            recv_sem=rs_recv_sem.at[d, s],
            device_id=(tgt,),
            device_id_type=pl.DeviceIdType.MESH,
        )
        rdma.start()
        return rdma

    def start_rs(s, src_t, src_b):
        return start_dir(0, right, s, src_t), start_dir(1, left, s, src_b)

    barrier = pltpu.get_barrier_semaphore()
    for nbr in (left, right):
        pl.semaphore_signal(
            barrier, inc=1, device_id=(nbr,), device_id_type=pl.DeviceIdType.MESH
        )
    pl.semaphore_wait(barrier, 2)

    cp0 = load_half(my, 0)
    cp1 = load_half(my, 1)
    cp0.wait()
    seed[0, :, :] = compute(0).astype(jnp.bfloat16)
    pend_t = start_dir(0, right, 0, seed.at[0])
    cp1.wait()
    seed[1, :, :] = compute(1).astype(jnp.bfloat16)
    pend_b = start_dir(1, left, 0, seed.at[1])

    pend = (pend_t, pend_b)
    for s in range(N_DEV - 1):
        c_t = lax.rem(my + (2 * N_DEV - 1 - s), N_DEV)
        c_b = lax.rem(my + s + 1, N_DEV)
        cp0 = load_half(c_t, 0)
        cp1 = load_half(c_b, 1)
        cp0.wait()
        p_t = compute(0)
        cp1.wait()
        p_b = compute(1)
        pend[0].wait()
        pend[1].wait()
        acc_t = rs_recv[0, s, :, :].astype(jnp.float32) + p_t
        acc_b = rs_recv[1, s, :, :].astype(jnp.float32) + p_b
        if s < N_DEV - 2:
            stage[0, s, :, :] = acc_t.astype(jnp.bfloat16)
            stage[1, s, :, :] = acc_b.astype(jnp.bfloat16)
            pend = start_rs(s + 1, stage.at[0, s], stage.at[1, s])
        else:
            stage[0, N_DEV - 2, :, :] = acc_t.astype(jnp.bfloat16)
            stage[1, N_DEV - 2, :, :] = acc_b.astype(jnp.bfloat16)

    out_dmas = []

    def store_out(src, c, d, j):
        cp = pltpu.make_async_copy(
            src, out_ref.at[pl.ds(c * blk + d * half, half), :], out_sem.at[d, j]
        )
        cp.start()
        out_dmas.append(cp)

    store_out(stage.at[0, N_DEV - 2], lax.rem(my + 1, N_DEV), 0, 0)
    store_out(stage.at[1, N_DEV - 2], left, 1, 0)

    for s in range(N_DEV - 1):
        srcs = (
            (stage.at[0, N_DEV - 2], stage.at[1, N_DEV - 2])
            if s == 0
            else (ag_recv.at[0, s - 1], ag_recv.at[1, s - 1])
        )
        rt = pltpu.make_async_remote_copy(
            src_ref=srcs[0],
            dst_ref=ag_recv.at[0, s],
            send_sem=ag_send_sem.at[0, s],
            recv_sem=ag_recv_sem.at[0, s],
            device_id=(right,),
            device_id_type=pl.DeviceIdType.MESH,
        )
        rb = pltpu.make_async_remote_copy(
            src_ref=srcs[1],
            dst_ref=ag_recv.at[1, s],
            send_sem=ag_send_sem.at[1, s],
            recv_sem=ag_recv_sem.at[1, s],
            device_id=(left,),
            device_id_type=pl.DeviceIdType.MESH,
        )
        rt.start()
        rb.start()
        rt.wait()
        rb.wait()
        c_t = lax.rem(my + N_DEV - s, N_DEV)
        c_b = lax.rem(my + s, N_DEV)
        store_out(ag_recv.at[0, s], c_t, 0, s + 1)
        store_out(ag_recv.at[1, s], c_b, 1, s + 1)

    for cp in out_dmas:
        cp.wait()


def _fused(x, w1, w2):
    m, k = x.shape
    n = w2.shape[1]
    half = m // N_DEV // 2
    return pl.pallas_call(
        _fused_body,
        out_shape=jax.ShapeDtypeStruct((m, n), jnp.bfloat16),
        in_specs=[
            pl.BlockSpec(memory_space=pl.ANY),
            pl.BlockSpec(memory_space=pltpu.VMEM),
            pl.BlockSpec(memory_space=pltpu.VMEM),
        ],
        out_specs=pl.BlockSpec(memory_space=pl.ANY),
        scratch_shapes=[
            pltpu.VMEM((2, half, k), jnp.float32),
            pltpu.VMEM((2, half, n), jnp.bfloat16),
            pltpu.VMEM((2, N_DEV - 1, half, n), jnp.bfloat16),
            pltpu.VMEM((2, N_DEV - 1, half, n), jnp.bfloat16),
            pltpu.VMEM((2, N_DEV - 1, half, n), jnp.bfloat16),
            pltpu.SemaphoreType.DMA((2,)),
            pltpu.SemaphoreType.DMA((2, N_DEV)),
            pltpu.SemaphoreType.DMA((2, N_DEV - 1)),
            pltpu.SemaphoreType.DMA((2, N_DEV - 1)),
            pltpu.SemaphoreType.DMA((2, N_DEV - 1)),
            pltpu.SemaphoreType.DMA((2, N_DEV - 1)),
        ],
        compiler_params=pltpu.CompilerParams(
            collective_id=0,
            vmem_limit_bytes=64 * 1024 * 1024,
        ),
    )(x, w1, w2)


def kernel(x, W1, W2):
    w1b = W1.astype(jnp.bfloat16)
    w2b = W2.astype(jnp.bfloat16)
    return _fused(x, w1b, w2b).astype(jnp.float32)


# device time: 176886 ns/iter; 1.6863x vs baseline; 1.0173x over previous
import jax

jax.config.update("jax_compilation_cache_dir", "/tmp/jax_comp_cache")
jax.config.update("jax_persistent_cache_min_compile_time_secs", 1)

import jax.numpy as jnp
from jax import lax
from jax.experimental import pallas as pl
from jax.experimental.pallas import tpu as pltpu

N_DEV = 4


_MLP_ROW_BLK = 256


def _mlp_body(x_ref, w1_ref, w2_ref, out_ref):
    h = jnp.dot(x_ref[:, :], w1_ref[:, :], preferred_element_type=jnp.float32)
    h = jnp.maximum(h, 0.0).astype(jnp.bfloat16)
    p = jnp.dot(h, w2_ref[:, :], preferred_element_type=jnp.float32)
    out_ref[:, :] = p.astype(jnp.bfloat16)


def _mlp(x, w1, w2):
    m, k = x.shape
    n = w2.shape[1]
    return pl.pallas_call(
        _mlp_body,
        grid=(m // _MLP_ROW_BLK,),
        out_shape=jax.ShapeDtypeStruct((m, n), jnp.bfloat16),
        in_specs=[
            pl.BlockSpec((_MLP_ROW_BLK, k), lambda i: (i, 0)),
            pl.BlockSpec(memory_space=pltpu.VMEM),
            pl.BlockSpec(memory_space=pltpu.VMEM),
        ],
        out_specs=pl.BlockSpec((_MLP_ROW_BLK, n), lambda i: (i, 0)),
        compiler_params=pltpu.CompilerParams(
            vmem_limit_bytes=64 * 1024 * 1024,
        ),
    )(x, w1, w2)


def _allreduce_body(
    p_ref,
    out_ref,
    rs_recv,
    stage,
    ag_recv,
    rs_send_sem,
    rs_recv_sem,
    ag_send_sem,
    ag_recv_sem,
):
    my = lax.axis_index("i")
    left = lax.rem(my + N_DEV - 1, N_DEV)
    right = lax.rem(my + 1, N_DEV)
    blk = p_ref.shape[0] // N_DEV
    half = blk // 2

    barrier = pltpu.get_barrier_semaphore()
    for nbr in (left, right):
        pl.semaphore_signal(
            barrier, inc=1, device_id=(nbr,), device_id_type=pl.DeviceIdType.MESH
        )
    pl.semaphore_wait(barrier, 2)

    def row0(c):
        return c * blk

    def row1(c):
        return c * blk + half

    stage[0, 0, :, :] = p_ref[pl.ds(row0(my), half), :]
    stage[1, 0, :, :] = p_ref[pl.ds(row1(my), half), :]

    def hop(src_refs, dst, send_sem, recv_sem, s):
        rdmas = []
        for d, tgt in ((0, right), (1, left)):
            rdma = pltpu.make_async_remote_copy(
                src_ref=src_refs[d],
                dst_ref=dst.at[d, s],
                send_sem=send_sem.at[d, s],
                recv_sem=recv_sem.at[d, s],
                device_id=(tgt,),
                device_id_type=pl.DeviceIdType.MESH,
            )
            rdma.start()
            rdmas.append(rdma)
        for rdma in rdmas:
            rdma.wait()

    for s in range(N_DEV - 1):
        hop((stage.at[0, s], stage.at[1, s]), rs_recv, rs_send_sem, rs_recv_sem, s)
        c_r = lax.rem(my + (2 * N_DEV - 1 - s), N_DEV)
        c_l = lax.rem(my + s + 1, N_DEV)
        acc_r = rs_recv[0, s, :, :].astype(jnp.float32) + p_ref[
            pl.ds(row0(c_r), half), :
        ].astype(jnp.float32)
        acc_l = rs_recv[1, s, :, :].astype(jnp.float32) + p_ref[
            pl.ds(row1(c_l), half), :
        ].astype(jnp.float32)
        if s < N_DEV - 2:
            stage[0, s + 1, :, :] = acc_r.astype(jnp.bfloat16)
            stage[1, s + 1, :, :] = acc_l.astype(jnp.bfloat16)
        else:
            out_ref[pl.ds(row0(c_r), half), :] = acc_r
            out_ref[pl.ds(row1(c_l), half), :] = acc_l
            stage[0, N_DEV - 1, :, :] = acc_r.astype(jnp.bfloat16)
            stage[1, N_DEV - 1, :, :] = acc_l.astype(jnp.bfloat16)

    for s in range(N_DEV - 1):
        if s == 0:
            srcs = (stage.at[0, N_DEV - 1], stage.at[1, N_DEV - 1])
        else:
            srcs = (ag_recv.at[0, s - 1], ag_recv.at[1, s - 1])
        hop(srcs, ag_recv, ag_send_sem, ag_recv_sem, s)
        c_r = lax.rem(my + (N_DEV - s), N_DEV)
        c_l = lax.rem(my + s, N_DEV)
        out_ref[pl.ds(row0(c_r), half), :] = ag_recv[0, s, :, :].astype(jnp.float32)
        out_ref[pl.ds(row1(c_l), half), :] = ag_recv[1, s, :, :].astype(jnp.float32)


def _allreduce(p):
    m, n = p.shape
    half = m // N_DEV // 2
    return pl.pallas_call(
        _allreduce_body,
        out_shape=jax.ShapeDtypeStruct((m, n), jnp.float32),
        in_specs=[pl.BlockSpec(memory_space=pltpu.VMEM)],
        out_specs=pl.BlockSpec(memory_space=pltpu.VMEM),
        scratch_shapes=[
            pltpu.VMEM((2, N_DEV - 1, half, n), jnp.bfloat16),
            pltpu.VMEM((2, N_DEV, half, n), jnp.bfloat16),
            pltpu.VMEM((2, N_DEV - 1, half, n), jnp.bfloat16),
            pltpu.SemaphoreType.DMA((2, N_DEV - 1)),
            pltpu.SemaphoreType.DMA((2, N_DEV - 1)),
            pltpu.SemaphoreType.DMA((2, N_DEV - 1)),
            pltpu.SemaphoreType.DMA((2, N_DEV - 1)),
        ],
        compiler_params=pltpu.CompilerParams(
            collective_id=0,
            vmem_limit_bytes=64 * 1024 * 1024,
        ),
    )(p)


def _fused_body(
    x_hbm,
    w1_ref,
    w2_ref,
    out_ref,
    xland,
    seed,
    stage,
    rs_recv,
    ag_recv,
    x_sem,
    out_sem,
    rs_send_sem,
    rs_recv_sem,
    ag_send_sem,
    ag_recv_sem,
):
    my = lax.axis_index("i")
    left = lax.rem(my + N_DEV - 1, N_DEV)
    right = lax.rem(my + 1, N_DEV)
    blk = out_ref.shape[0] // N_DEV
    half = blk // 2

    def load_half(c, d):
        cp = pltpu.make_async_copy(
            x_hbm.at[pl.ds(c * blk + d * half, half), :], xland.at[d], x_sem.at[d]
        )
        cp.start()
        return cp

    def compute(d):
        xb = xland[d, :, :].astype(jnp.bfloat16)
        h = jnp.dot(xb, w1_ref[:, :], preferred_element_type=jnp.float32)
        h = jnp.maximum(h, 0.0).astype(jnp.bfloat16)
        return jnp.dot(h, w2_ref[:, :], preferred_element_type=jnp.float32)

    def start_dir(d, tgt, s, src):
        rdma = pltpu.make_async_remote_copy(
            src_ref=src,
            dst_ref=rs_recv.at[d, s],
            send_sem=rs_send_sem.at[d, s],
            recv_sem=rs_recv_sem.at[d, s],
            device_id=(tgt,),
            device_id_type=pl.DeviceIdType.MESH,
        )
        rdma.start()
        return rdma

    def start_rs(s, src_t, src_b):
        return start_dir(0, right, s, src_t), start_dir(1, left, s, src_b)

    barrier = pltpu.get_barrier_semaphore()
    for nbr in (left, right):
        pl.semaphore_signal(
            barrier, inc=1, device_id=(nbr,), device_id_type=pl.DeviceIdType.MESH
        )
    pl.semaphore_wait(barrier, 2)

    cp0 = load_half(my, 0)
    cp1 = load_half(my, 1)
    cp0.wait()
    seed[0, :, :] = compute(0).astype(jnp.bfloat16)
    pend_t = start_dir(0, right, 0, seed.at[0])
    cp1.wait()
    seed[1, :, :] = compute(1).astype(jnp.bfloat16)
    pend_b = start_dir(1, left, 0, seed.at[1])

    pend = (pend_t, pend_b)
    for s in range(N_DEV - 1):
        c_t = lax.rem(my + (2 * N_DEV - 1 - s), N_DEV)
        c_b = lax.rem(my + s + 1, N_DEV)
        cp0 = load_half(c_t, 0)
        cp1 = load_half(c_b, 1)
        cp0.wait()
        p_t = compute(0)
        cp1.wait()
        p_b = compute(1)
        pend[0].wait()
        pend[1].wait()
        acc_t = rs_recv[0, s, :, :].astype(jnp.float32) + p_t
        acc_b = rs_recv[1, s, :, :].astype(jnp.float32) + p_b
        if s < N_DEV - 2:
            stage[0, s, :, :] = acc_t.astype(jnp.bfloat16)
            stage[1, s, :, :] = acc_b.astype(jnp.bfloat16)
            pend = start_rs(s + 1, stage.at[0, s], stage.at[1, s])
        else:
            stage[0, N_DEV - 2, :, :] = acc_t.astype(jnp.bfloat16)
            stage[1, N_DEV - 2, :, :] = acc_b.astype(jnp.bfloat16)

    out_dmas = []

    def store_out(src, c, d, j):
        cp = pltpu.make_async_copy(
            src, out_ref.at[pl.ds(c * blk + d * half, half), :], out_sem.at[d, j]
        )
        cp.start()
        out_dmas.append(cp)

    store_out(stage.at[0, N_DEV - 2], lax.rem(my + 1, N_DEV), 0, 0)
    store_out(stage.at[1, N_DEV - 2], left, 1, 0)

    for s in range(N_DEV - 1):
        srcs = (
            (stage.at[0, N_DEV - 2], stage.at[1, N_DEV - 2])
            if s == 0
            else (ag_recv.at[0, s - 1], ag_recv.at[1, s - 1])
        )
        rt = pltpu.make_async_remote_copy(
            src_ref=srcs[0],
            dst_ref=ag_recv.at[0, s],
            send_sem=ag_send_sem.at[0, s],
            recv_sem=ag_recv_sem.at[0, s],
            device_id=(right,),
            device_id_type=pl.DeviceIdType.MESH,
        )
        rb = pltpu.make_async_remote_copy(
            src_ref=srcs[1],
            dst_ref=ag_recv.at[1, s],
            send_sem=ag_send_sem.at[1, s],
            recv_sem=ag_recv_sem.at[1, s],
            device_id=(left,),
            device_id_type=pl.DeviceIdType.MESH,
        )
        rt.start()
        rb.start()
        rt.wait()
        rb.wait()
        c_t = lax.rem(my + N_DEV - s, N_DEV)
        c_b = lax.rem(my + s, N_DEV)
        store_out(ag_recv.at[0, s], c_t, 0, s + 1)
        store_out(ag_recv.at[1, s], c_b, 1, s + 1)

    for cp in out_dmas:
        cp.wait()


def _fused(x, w1, w2):
    m, k = x.shape
    n = w2.shape[1]
    half = m // N_DEV // 2
    return pl.pallas_call(
        _fused_body,
        out_shape=jax.ShapeDtypeStruct((m, n), jnp.bfloat16),
        in_specs=[
            pl.BlockSpec(memory_space=pl.ANY),
            pl.BlockSpec(memory_space=pltpu.VMEM),
            pl.BlockSpec(memory_space=pltpu.VMEM),
        ],
        out_specs=pl.BlockSpec(memory_space=pl.ANY),
        scratch_shapes=[
            pltpu.VMEM((2, half, k), jnp.float32),
            pltpu.VMEM((2, half, n), jnp.bfloat16),
            pltpu.VMEM((2, N_DEV - 1, half, n), jnp.bfloat16),
            pltpu.VMEM((2, N_DEV - 1, half, n), jnp.bfloat16),
            pltpu.VMEM((2, N_DEV - 1, half, n), jnp.bfloat16),
            pltpu.SemaphoreType.DMA((2,)),
            pltpu.SemaphoreType.DMA((2, N_DEV)),
            pltpu.SemaphoreType.DMA((2, N_DEV - 1)),
            pltpu.SemaphoreType.DMA((2, N_DEV - 1)),
            pltpu.SemaphoreType.DMA((2, N_DEV - 1)),
            pltpu.SemaphoreType.DMA((2, N_DEV - 1)),
        ],
        compiler_params=pltpu.CompilerParams(
            collective_id=0,
            vmem_limit_bytes=64 * 1024 * 1024,
        ),
    )(x, w1, w2)


def kernel(x, W1, W2):
    w1b = W1.astype(jnp.bfloat16)
    w2b = W2.astype(jnp.bfloat16)
    return _fused(x, w1b, w2b)
